# baseline (speedup 1.0000x reference)
import functools

import jax
import jax.numpy as jnp
from jax import lax
from jax.experimental import pallas as pl
from jax.experimental.pallas import tpu as pltpu
from jax.experimental.pallas import tpu_sc as plsc

KK = 15
DD = 3
EPSG = 1e-15
EDGE_BLK = 2000

CHUNK = 128
NC, NS = 2, 16
NW = NC * NS
ZROWS = 1000
FW = 128


def _sc_mesh():
    return plsc.VectorSubcoreMesh(core_axis_name="c", subcore_axis_name="s")


def _gather_body(nch, f_in, h_hbm, src_hbm, out_hbm, idx_v, rows_v, sem):
    wid = lax.axis_index("s") * NC + lax.axis_index("c")
    nj = (nch + NW - 1) // NW

    def body(j, carry):
        cj = wid + NW * j

        @pl.when(cj < nch)
        def _():
            pltpu.sync_copy(src_hbm.at[cj], idx_v)
            pltpu.async_copy(h_hbm.at[idx_v], rows_v, sem).wait()
            pltpu.sync_copy(rows_v, out_hbm.at[pl.ds(cj * CHUNK, CHUNK)])

        return carry

    lax.fori_loop(0, nj, body, 0)


def _sc_gather(h, src2):
    n, f_in = h.shape
    nch = src2.shape[0]
    k = pl.kernel(
        functools.partial(_gather_body, nch, f_in),
        out_type=jax.ShapeDtypeStruct((nch * CHUNK, f_in), jnp.float32),
        mesh=_sc_mesh(),
        scratch_types=[
            pltpu.VMEM((CHUNK,), jnp.int32),
            pltpu.VMEM((CHUNK, f_in), jnp.float32),
            pltpu.SemaphoreType.DMA,
        ],
    )
    return k(h, src2)


def _scatter_body(n, do_cnt, msg_hbm, dst_hbm, zero_hbm, zero1_hbm, *refs):
    if do_cnt:
        parts_hbm, cnt_hbm, idx_v, vals_v, ones_v, acc_s, cnt_s = refs
    else:
        parts_hbm, idx_v, vals_v, acc_s = refs
    cid = lax.axis_index("c")
    sid = lax.axis_index("s")
    wid = sid * NC + cid
    nch = msg_hbm.shape[0] // CHUNK
    nslab = n // ZROWS

    @pl.when(sid < nslab)
    def _():
        pltpu.sync_copy(zero_hbm, acc_s.at[pl.ds(sid * ZROWS, ZROWS)])

    if do_cnt:
        ncs = (n + 1023) // 1024
        @pl.when(sid == nslab)
        def _():
            for t in range(ncs):
                pltpu.sync_copy(zero1_hbm, cnt_s.at[pl.ds(t * 1024, 1024)])
        for t in range(CHUNK // 16):
            ones_v[pl.ds(t * 16, 16)] = jnp.ones((16,), jnp.float32)
    plsc.subcore_barrier()

    nj = (nch + NW - 1) // NW

    def body(j, carry):
        cj = wid + NW * j

        @pl.when(cj < nch)
        def _():
            pltpu.sync_copy(dst_hbm.at[cj], idx_v)
            pltpu.sync_copy(msg_hbm.at[pl.ds(cj * CHUNK, CHUNK)], vals_v)
            pltpu.sync_copy(vals_v, acc_s.at[idx_v], add=True)
            if do_cnt:
                pltpu.sync_copy(ones_v, cnt_s.at[idx_v], add=True)

        return carry

    lax.fori_loop(0, nj, body, 0)
    plsc.subcore_barrier()

    @pl.when(sid < nslab)
    def _():
        rows = pl.ds(sid * ZROWS, ZROWS)
        pltpu.sync_copy(acc_s.at[rows], parts_hbm.at[cid].at[rows])

    if do_cnt:
        ncs = (n + 1023) // 1024
        @pl.when(sid == nslab)
        def _():
            for t in range(ncs):
                rows = pl.ds(t * 1024, 1024)
                pltpu.sync_copy(cnt_s.at[rows], cnt_hbm.at[cid].at[rows])


def _sc_scatter(msg, dst2, n, do_cnt):
    e, f_out = msg.shape
    zero = jnp.zeros((ZROWS, f_out), jnp.float32)
    zero1 = jnp.zeros((1024,), jnp.float32)
    out_type = [jax.ShapeDtypeStruct((NC, n, f_out), jnp.float32)]
    scratch = [
        pltpu.VMEM((CHUNK,), jnp.int32),
        pltpu.VMEM((CHUNK, f_out), jnp.float32),
    ]
    if do_cnt:
        npad = ((n + 1023) // 1024) * 1024
        out_type.append(jax.ShapeDtypeStruct((NC, npad), jnp.float32))
        scratch.append(pltpu.VMEM((CHUNK,), jnp.float32))
    scratch.append(pltpu.VMEM_SHARED((n, f_out), jnp.float32))
    if do_cnt:
        scratch.append(pltpu.VMEM_SHARED((((n + 1023) // 1024) * 1024,), jnp.float32))
    k = pl.kernel(
        functools.partial(_scatter_body, n, do_cnt),
        out_type=tuple(out_type) if do_cnt else out_type[0],
        mesh=_sc_mesh(),
        scratch_types=scratch,
    )
    return k(msg, dst2, zero, zero1)


def _edge_body(attr_ref, xe_ref, mu_ref, alpha_ref, g_ref, msg_ref, z_ref):
    eb = attr_ref.shape[0]
    acc_g = jnp.zeros((eb, KK), dtype=jnp.float32)
    for d in range(DD):
        col = attr_ref[:, d : d + 1]
        diff = col - mu_ref[d : d + 1, :]
        acc_g = acc_g + diff * diff * alpha_ref[d : d + 1, :]
    gauss = jnp.exp(acc_g)
    xe = xe_ref[...]
    for k in range(KK):
        z_ref[:, k * FW : (k + 1) * FW] = xe * gauss[:, k : k + 1]
    msg_ref[...] = jnp.dot(z_ref[...], g_ref[...],
                           preferred_element_type=jnp.float32)


def _edge_msgs(edge_attr, xe, mu_t, alpha_t, g3):
    e = xe.shape[0]
    f_in = xe.shape[1]
    f_out = g3.shape[2]
    g_flat = g3.reshape(KK * f_in, f_out)
    assert e % EDGE_BLK == 0
    grid = e // EDGE_BLK
    return pl.pallas_call(
        _edge_body,
        grid=(grid,),
        in_specs=[
            pl.BlockSpec((EDGE_BLK, DD), lambda i: (i, 0)),
            pl.BlockSpec((EDGE_BLK, f_in), lambda i: (i, 0)),
            pl.BlockSpec((DD, KK), lambda i: (0, 0)),
            pl.BlockSpec((DD, KK), lambda i: (0, 0)),
            pl.BlockSpec((KK * f_in, f_out), lambda i: (0, 0)),
        ],
        out_specs=pl.BlockSpec((EDGE_BLK, f_out), lambda i: (i, 0)),
        out_shape=jax.ShapeDtypeStruct((e, f_out), jnp.float32),
        scratch_shapes=[pltpu.VMEM((EDGE_BLK, KK * f_in), jnp.float32)],
    )(edge_attr, xe, mu_t, alpha_t, g_flat)


def _node_body(do_act, parts_ref, cnt_ref, x_ref, root_ref, bias_ref,
               gamma_ref, beta_ref, out_ref):
    agg = (parts_ref[0] + parts_ref[1]) / jnp.maximum(cnt_ref[...], 1.0)
    r = jnp.dot(x_ref[...], root_ref[...], preferred_element_type=jnp.float32)
    h = agg + r + bias_ref[...]
    if do_act:
        h = jnp.where(h > 0, h, jnp.exp(h) - 1.0)
        m = jnp.mean(h, axis=0, keepdims=True)
        c = h - m
        v = jnp.mean(c * c, axis=0, keepdims=True)
        h = c / jnp.sqrt(v + 1e-5) * gamma_ref[...] + beta_ref[...]
    out_ref[...] = h


def _node_update(parts, cnt, x, root, bias, gamma, beta, do_act):
    n = x.shape[0]
    f_out = root.shape[1]
    return pl.pallas_call(
        functools.partial(_node_body, do_act),
        out_shape=jax.ShapeDtypeStruct((n, f_out), jnp.float32),
    )(parts, cnt, x, root, bias, gamma, beta)


def _padw(a, w=FW):
    pad = [(0, 0)] * (a.ndim - 1) + [(0, w - a.shape[-1])]
    return jnp.pad(a, pad)


def kernel(x, edge_index, edge_attr, params):
    n = x.shape[0]
    e = edge_attr.shape[0]
    f_final = params["conv4"]["root"].shape[1]
    assert e % CHUNK == 0
    nch = e // CHUNK
    src2 = edge_index[0].astype(jnp.int32).reshape(nch, CHUNK)
    dst2 = edge_index[1].astype(jnp.int32).reshape(nch, CHUNK)
    h = _padw(x)
    cnt = None
    names = ("conv1", "conv2", "conv3", "conv4")
    bns = ("bn1", "bn2", "bn3", None)
    for name, bn in zip(names, bns):
        p = params[name]
        f_in, f_out = p["root"].shape
        mu_t = p["mu"].T
        alpha_t = (-0.5 / (EPSG + p["sigma"] ** 2)).T
        g3 = p["g"].reshape(f_in, KK, f_out).transpose(1, 0, 2)
        g3 = jnp.pad(g3, ((0, 0), (0, FW - f_in), (0, FW - f_out)))
        xe = _sc_gather(h, src2)
        msg = _edge_msgs(edge_attr, xe, mu_t, alpha_t, g3)
        if cnt is None:
            parts, cnt2 = _sc_scatter(msg, dst2, n, do_cnt=True)
            cnt = (cnt2[0, :n] + cnt2[1, :n]).reshape(n, 1)
        else:
            parts = _sc_scatter(msg, dst2, n, do_cnt=False)
        if bn is None:
            gamma = jnp.ones((1, FW), jnp.float32)
            beta = jnp.zeros((1, FW), jnp.float32)
        else:
            gamma = _padw(params[bn]["gamma"].reshape(1, f_out))
            beta = _padw(params[bn]["beta"].reshape(1, f_out))
        h = _node_update(parts, cnt, h, jnp.pad(p["root"], ((0, FW - f_in), (0, FW - f_out))),
                         _padw(p["bias"].reshape(1, f_out)), gamma, beta,
                         do_act=bn is not None)
    return h[:, :f_final]


# speedup vs baseline: 1.1257x; 1.1257x over previous
import functools

import jax
import jax.numpy as jnp
from jax import lax
from jax.experimental import pallas as pl
from jax.experimental.pallas import tpu as pltpu
from jax.experimental.pallas import tpu_sc as plsc

KK = 15
DD = 3
EPSG = 1e-15
EDGE_BLK = 2000

CHUNK = 128
NC, NS = 2, 16
NW = NC * NS
ZROWS = 1000
FW = 128


def _sc_mesh():
    return plsc.VectorSubcoreMesh(core_axis_name="c", subcore_axis_name="s")


def _gather_body(nch, f_in, h_hbm, src_hbm, out_hbm, idx_v, rows_v, sem):
    wid = lax.axis_index("s") * NC + lax.axis_index("c")
    nj = (nch + NW - 1) // NW

    def body(j, carry):
        cj = wid + NW * j

        @pl.when(cj < nch)
        def _():
            pltpu.sync_copy(src_hbm.at[cj], idx_v)
            pltpu.async_copy(h_hbm.at[idx_v], rows_v, sem).wait()
            pltpu.sync_copy(rows_v, out_hbm.at[pl.ds(cj * CHUNK, CHUNK)])

        return carry

    lax.fori_loop(0, nj, body, 0)


def _sc_gather(h, src2):
    n, f_in = h.shape
    nch = src2.shape[0]
    k = pl.kernel(
        functools.partial(_gather_body, nch, f_in),
        out_type=jax.ShapeDtypeStruct((nch * CHUNK, f_in), jnp.float32),
        mesh=_sc_mesh(),
        scratch_types=[
            pltpu.VMEM((CHUNK,), jnp.int32),
            pltpu.VMEM((CHUNK, f_in), jnp.float32),
            pltpu.SemaphoreType.DMA,
        ],
    )
    return k(h, src2)


def _scatter_body(n, do_cnt, msg_hbm, dst_hbm, zero_hbm, zero1_hbm, *refs):
    if do_cnt:
        parts_hbm, cnt_hbm, idx_v, vals_v, ones_v, acc_s, cnt_s = refs
    else:
        parts_hbm, idx_v, vals_v, acc_s = refs
    cid = lax.axis_index("c")
    sid = lax.axis_index("s")
    wid = sid * NC + cid
    nch = msg_hbm.shape[0] // CHUNK
    nslab = n // ZROWS

    @pl.when(sid < nslab)
    def _():
        pltpu.sync_copy(zero_hbm, acc_s.at[pl.ds(sid * ZROWS, ZROWS)])

    if do_cnt:
        ncs = (n + 1023) // 1024
        @pl.when(sid == nslab)
        def _():
            for t in range(ncs):
                pltpu.sync_copy(zero1_hbm, cnt_s.at[pl.ds(t * 1024, 1024)])
        for t in range(CHUNK // 16):
            ones_v[pl.ds(t * 16, 16)] = jnp.ones((16,), jnp.float32)
    plsc.subcore_barrier()

    nj = (nch + NW - 1) // NW

    def body(j, carry):
        cj = wid + NW * j

        @pl.when(cj < nch)
        def _():
            pltpu.sync_copy(dst_hbm.at[cj], idx_v)
            pltpu.sync_copy(msg_hbm.at[pl.ds(cj * CHUNK, CHUNK)], vals_v)
            pltpu.sync_copy(vals_v, acc_s.at[idx_v], add=True)
            if do_cnt:
                pltpu.sync_copy(ones_v, cnt_s.at[idx_v], add=True)

        return carry

    lax.fori_loop(0, nj, body, 0)
    plsc.subcore_barrier()

    @pl.when(sid < nslab)
    def _():
        rows = pl.ds(sid * ZROWS, ZROWS)
        pltpu.sync_copy(acc_s.at[rows], parts_hbm.at[cid].at[rows])

    if do_cnt:
        ncs = (n + 1023) // 1024
        @pl.when(sid == nslab)
        def _():
            for t in range(ncs):
                rows = pl.ds(t * 1024, 1024)
                pltpu.sync_copy(cnt_s.at[rows], cnt_hbm.at[cid].at[rows])


def _sc_scatter(msg, dst2, n, do_cnt):
    e, f_out = msg.shape
    zero = jnp.zeros((ZROWS, f_out), jnp.float32)
    zero1 = jnp.zeros((1024,), jnp.float32)
    out_type = [jax.ShapeDtypeStruct((NC, n, f_out), jnp.float32)]
    scratch = [
        pltpu.VMEM((CHUNK,), jnp.int32),
        pltpu.VMEM((CHUNK, f_out), jnp.float32),
    ]
    if do_cnt:
        npad = ((n + 1023) // 1024) * 1024
        out_type.append(jax.ShapeDtypeStruct((NC, npad), jnp.float32))
        scratch.append(pltpu.VMEM((CHUNK,), jnp.float32))
    scratch.append(pltpu.VMEM_SHARED((n, f_out), jnp.float32))
    if do_cnt:
        scratch.append(pltpu.VMEM_SHARED((((n + 1023) // 1024) * 1024,), jnp.float32))
    k = pl.kernel(
        functools.partial(_scatter_body, n, do_cnt),
        out_type=tuple(out_type) if do_cnt else out_type[0],
        mesh=_sc_mesh(),
        scratch_types=scratch,
    )
    return k(msg, dst2, zero, zero1)


def _edge_body(attr_ref, xe_ref, mu_ref, alpha_ref, g_ref, msg_ref, z_ref):
    eb = attr_ref.shape[0]
    acc_g = jnp.zeros((eb, KK), dtype=jnp.float32)
    for d in range(DD):
        col = attr_ref[:, d : d + 1]
        diff = col - mu_ref[d : d + 1, :]
        acc_g = acc_g + diff * diff * alpha_ref[d : d + 1, :]
    gauss = jnp.exp(acc_g).astype(jnp.bfloat16)
    xe = xe_ref[...].astype(jnp.bfloat16)
    for k in range(KK):
        z_ref[:, k * FW : (k + 1) * FW] = xe * gauss[:, k : k + 1]
    msg_ref[...] = jnp.dot(z_ref[...], g_ref[...],
                           preferred_element_type=jnp.float32)


def _edge_msgs(edge_attr, xe, mu_t, alpha_t, g3):
    e = xe.shape[0]
    f_in = xe.shape[1]
    f_out = g3.shape[2]
    g_flat = g3.reshape(KK * f_in, f_out).astype(jnp.bfloat16)
    assert e % EDGE_BLK == 0
    grid = e // EDGE_BLK
    return pl.pallas_call(
        _edge_body,
        grid=(grid,),
        in_specs=[
            pl.BlockSpec((EDGE_BLK, DD), lambda i: (i, 0)),
            pl.BlockSpec((EDGE_BLK, f_in), lambda i: (i, 0)),
            pl.BlockSpec((DD, KK), lambda i: (0, 0)),
            pl.BlockSpec((DD, KK), lambda i: (0, 0)),
            pl.BlockSpec((KK * f_in, f_out), lambda i: (0, 0)),
        ],
        out_specs=pl.BlockSpec((EDGE_BLK, f_out), lambda i: (i, 0)),
        out_shape=jax.ShapeDtypeStruct((e, f_out), jnp.float32),
        scratch_shapes=[pltpu.VMEM((EDGE_BLK, KK * f_in), jnp.bfloat16)],
    )(edge_attr, xe, mu_t, alpha_t, g_flat)


def _node_body(do_act, parts_ref, cnt_ref, x_ref, root_ref, bias_ref,
               gamma_ref, beta_ref, out_ref):
    agg = (parts_ref[0] + parts_ref[1]) / jnp.maximum(cnt_ref[...], 1.0)
    r = jnp.dot(x_ref[...], root_ref[...], preferred_element_type=jnp.float32)
    h = agg + r + bias_ref[...]
    if do_act:
        h = jnp.where(h > 0, h, jnp.exp(h) - 1.0)
        m = jnp.mean(h, axis=0, keepdims=True)
        c = h - m
        v = jnp.mean(c * c, axis=0, keepdims=True)
        h = c / jnp.sqrt(v + 1e-5) * gamma_ref[...] + beta_ref[...]
    out_ref[...] = h


def _node_update(parts, cnt, x, root, bias, gamma, beta, do_act):
    n = x.shape[0]
    f_out = root.shape[1]
    return pl.pallas_call(
        functools.partial(_node_body, do_act),
        out_shape=jax.ShapeDtypeStruct((n, f_out), jnp.float32),
    )(parts, cnt, x, root, bias, gamma, beta)


def _padw(a, w=FW):
    pad = [(0, 0)] * (a.ndim - 1) + [(0, w - a.shape[-1])]
    return jnp.pad(a, pad)


def kernel(x, edge_index, edge_attr, params):
    n = x.shape[0]
    e = edge_attr.shape[0]
    f_final = params["conv4"]["root"].shape[1]
    assert e % CHUNK == 0
    nch = e // CHUNK
    src2 = edge_index[0].astype(jnp.int32).reshape(nch, CHUNK)
    dst2 = edge_index[1].astype(jnp.int32).reshape(nch, CHUNK)
    h = _padw(x)
    cnt = None
    names = ("conv1", "conv2", "conv3", "conv4")
    bns = ("bn1", "bn2", "bn3", None)
    for name, bn in zip(names, bns):
        p = params[name]
        f_in, f_out = p["root"].shape
        mu_t = p["mu"].T
        alpha_t = (-0.5 / (EPSG + p["sigma"] ** 2)).T
        g3 = p["g"].reshape(f_in, KK, f_out).transpose(1, 0, 2)
        g3 = jnp.pad(g3, ((0, 0), (0, FW - f_in), (0, FW - f_out)))
        xe = _sc_gather(h, src2)
        msg = _edge_msgs(edge_attr, xe, mu_t, alpha_t, g3)
        if cnt is None:
            parts, cnt2 = _sc_scatter(msg, dst2, n, do_cnt=True)
            cnt = (cnt2[0, :n] + cnt2[1, :n]).reshape(n, 1)
        else:
            parts = _sc_scatter(msg, dst2, n, do_cnt=False)
        if bn is None:
            gamma = jnp.ones((1, FW), jnp.float32)
            beta = jnp.zeros((1, FW), jnp.float32)
        else:
            gamma = _padw(params[bn]["gamma"].reshape(1, f_out))
            beta = _padw(params[bn]["beta"].reshape(1, f_out))
        h = _node_update(parts, cnt, h, jnp.pad(p["root"], ((0, FW - f_in), (0, FW - f_out))),
                         _padw(p["bias"].reshape(1, f_out)), gamma, beta,
                         do_act=bn is not None)
    return h[:, :f_final]
